# Initial kernel scaffold; baseline (speedup 1.0000x reference)
import jax
import jax.numpy as jnp
from jax.experimental import pallas as pl


def kernel(rpn_cls_score, gt_boxes, im_info):
    raise NotImplementedError("write your pallas kernel here")


# speedup vs baseline: 29.3052x; 29.3052x over previous
import numpy as np
import jax
import jax.numpy as jnp
from jax.experimental import pallas as pl
from jax.experimental.pallas import tpu as pltpu

_FEAT_STRIDE = 16
_RPN_NEG = 0.3
_RPN_POS = 0.7
_RPN_BATCH = 256.0
_NUM_FG = 128.0
_H = 64
_W = 64
_A = 9
_N = _H * _W * _A
_R = _N // 128
_C = 128
_G = 100


def _np_all_anchors():
    base_size = 16.0
    xc = yc = 0.5 * (base_size - 1.0)
    size = base_size * base_size
    rows = []
    for r in (0.5, 1.0, 2.0):
        ws = np.round(np.sqrt(size / r))
        hs = np.round(ws * r)
        for s in (8.0, 16.0, 32.0):
            W_ = ws * s
            H_ = hs * s
            rows.append([xc - 0.5 * (W_ - 1.0), yc - 0.5 * (H_ - 1.0),
                         xc + 0.5 * (W_ - 1.0), yc + 0.5 * (H_ - 1.0)])
    base = np.array(rows, dtype=np.float32)
    sx = np.arange(_W, dtype=np.float32) * _FEAT_STRIDE
    sy = np.arange(_H, dtype=np.float32) * _FEAT_STRIDE
    SY, SX = np.meshgrid(sy, sx, indexing="ij")
    shifts = np.stack([SX.ravel(), SY.ravel(), SX.ravel(), SY.ravel()], axis=1)
    allv = (base[None, :, :] + shifts[:, None, :]).reshape(_N, 4)
    return allv.astype(np.float32)


_ANCHORS = _np_all_anchors()
_AM = np.ascontiguousarray(
    _ANCHORS.reshape(_H * _W, _A, 4).transpose(1, 0, 2).reshape(_N, 4))
_AX1 = np.ascontiguousarray(_AM[:, 0].reshape(_R, _C))
_AY1 = np.ascontiguousarray(_AM[:, 1].reshape(_R, _C))
_AX2 = np.ascontiguousarray(_AM[:, 2].reshape(_R, _C))
_AY2 = np.ascontiguousarray(_AM[:, 3].reshape(_R, _C))
_AX2P = _AX2 + 1.0
_AY2P = _AY2 + 1.0


def _body(gx1_ref, gy1_ref, gx2_ref, gy2_ref, areag_ref, im_ref,
          ax1_ref, ay1_ref, ax2_ref, ay2_ref, ax2p_ref, ay2p_ref,
          lab_ref, dx_ref, dy_ref, dw_ref, dh_ref, biw_ref, bow_ref,
          ovs_ref):
    ax1 = ax1_ref[...]
    ay1 = ay1_ref[...]
    ax2 = ax2_ref[...]
    ay2 = ay2_ref[...]
    ax2p = ax2p_ref[...]
    ay2p = ay2p_ref[...]
    im_h = im_ref[0]
    im_w = im_ref[1]
    inside = (ax1 >= 0.0) & (ay1 >= 0.0) & (ax2 < im_w) & (ay2 < im_h)
    area_a = (ax2p - ax1) * (ay2p - ay1)

    def pass1(j, carry):
        mx, g1, g2, g3, g4 = carry
        gx1 = gx1_ref[j]
        gy1 = gy1_ref[j]
        gx2 = gx2_ref[j]
        gy2 = gy2_ref[j]
        iw = jnp.minimum(ax2, gx2) - jnp.maximum(ax1, gx1) + 1.0
        ih = jnp.minimum(ay2, gy2) - jnp.maximum(ay1, gy1) + 1.0
        iw = jnp.maximum(iw, 0.0)
        ih = jnp.maximum(ih, 0.0)
        inter = iw * ih
        ov = jnp.where(inside, inter / (area_a + areag_ref[j] - inter), -1.0)
        ovs_ref[j] = ov
        upd = ov > mx
        mx = jnp.where(upd, ov, mx)
        g1 = jnp.where(upd, gx1, g1)
        g2 = jnp.where(upd, gy1, g2)
        g3 = jnp.where(upd, gx2, g3)
        g4 = jnp.where(upd, gy2, g4)
        return (mx, g1, g2, g3, g4)

    zero = jnp.zeros((_R, _C), jnp.float32)
    init = (jnp.full((_R, _C), -jnp.inf, jnp.float32), zero, zero, zero, zero)
    mx, g1, g2, g3, g4 = jax.lax.fori_loop(0, _G, pass1, init, unroll=4)

    ovs = ovs_ref[...]
    gtm = jnp.max(ovs, axis=(1, 2), keepdims=True)
    eq = jnp.where(ovs == gtm, 1.0, 0.0)
    isgt = (jnp.max(eq, axis=0) > 0.0) & inside

    lab = jnp.full((_R, _C), -1.0, jnp.float32)
    lab = jnp.where(mx < _RPN_NEG, 0.0, lab)
    lab = jnp.where(isgt, 1.0, lab)
    lab = jnp.where(mx >= _RPN_POS, 1.0, lab)
    lab = jnp.where(inside, lab, -1.0)

    _BR = (_H * _W) // _C
    ki = jax.lax.broadcasted_iota(jnp.int32, (_C, _C), 0)
    kj = jax.lax.broadcasted_iota(jnp.int32, (_C, _C), 1)
    upper = (ki <= kj).astype(jnp.float32)
    ri = jax.lax.broadcasted_iota(jnp.int32, (_R, _R), 0)
    rj = jax.lax.broadcasted_iota(jnp.int32, (_R, _R), 1)
    lower_blk = ((rj < ri) & (rj // _BR == ri // _BR)).astype(jnp.float32)

    def cumsum_orig_order(m):
        cw = jnp.dot(m, upper, preferred_element_type=jnp.float32)
        rs = cw[:, _C - 1:_C]
        pre = jnp.dot(lower_blk, rs, preferred_element_type=jnp.float32)
        s = cw + pre
        t = s[0:_BR, :]
        for a in range(1, _A):
            t = t + s[a * _BR:(a + 1) * _BR, :]
        out = [None] * _A
        u = jnp.zeros((_BR, _C), jnp.float32)
        for a in range(_A - 1, -1, -1):
            out[a] = t - u
            u = u + m[a * _BR:(a + 1) * _BR, :]
        return jnp.concatenate(out, axis=0)
    cumsum_flat = cumsum_orig_order

    fg = lab == 1.0
    cfg = cumsum_flat(fg.astype(jnp.float32))
    lab = jnp.where(fg & (cfg > _NUM_FG), -1.0, lab)
    num_bg = _RPN_BATCH - jnp.sum((lab == 1.0).astype(jnp.float32))
    bg = lab == 0.0
    cbg = cumsum_flat(bg.astype(jnp.float32))
    lab = jnp.where(bg & (cbg > num_bg), -1.0, lab)

    lab_ref[...] = lab

    ew = ax2p - ax1
    eh = ay2p - ay1
    ecx = ax1 + 0.5 * ew
    ecy = ay1 + 0.5 * eh
    gw = g3 - g1 + 1.0
    gh = g4 - g2 + 1.0
    gcx = g1 + 0.5 * gw
    gcy = g2 + 0.5 * gh
    dx_ref[...] = jnp.where(inside, (gcx - ecx) / ew, 0.0)
    dy_ref[...] = jnp.where(inside, (gcy - ecy) / eh, 0.0)
    dw_ref[...] = jnp.where(inside, jnp.log(gw / ew), 0.0)
    dh_ref[...] = jnp.where(inside, jnp.log(gh / eh), 0.0)

    biw_ref[...] = jnp.where(lab == 1.0, 1.0, 0.0)
    num_examples = jnp.sum((lab >= 0.0).astype(jnp.float32))
    bow_ref[...] = jnp.where(lab >= 0.0, 1.0 / num_examples, 0.0)


_plane = jax.ShapeDtypeStruct((_R, _C), jnp.float32)
_smem_spec = pl.BlockSpec(memory_space=pltpu.SMEM)

_call = pl.pallas_call(
    _body,
    out_shape=(_plane,) * 7,
    in_specs=[_smem_spec] * 6 + [pl.BlockSpec((_R, _C), lambda: (0, 0))] * 6,
    out_specs=tuple(pl.BlockSpec((_R, _C), lambda: (0, 0)) for _ in range(7)),
    scratch_shapes=[pltpu.VMEM((_G, _R, _C), jnp.float32)],
)


def kernel(rpn_cls_score, gt_boxes, im_info):
    gx1 = gt_boxes[:, 0]
    gy1 = gt_boxes[:, 1]
    gx2 = gt_boxes[:, 2]
    gy2 = gt_boxes[:, 3]
    areag = (gx2 - gx1 + 1.0) * (gy2 - gy1 + 1.0)
    im = im_info[0, :2]
    lab, dx, dy, dw, dh, biw, bow = _call(
        gx1, gy1, gx2, gy2, areag, im,
        _AX1, _AY1, _AX2, _AY2, _AX2P, _AY2P)

    labels_out = lab.reshape(1, 1, _A * _H, _W)

    hw = _H * _W
    bt_out = jnp.stack([dx.reshape(_A, hw), dy.reshape(_A, hw),
                        dw.reshape(_A, hw), dh.reshape(_A, hw)],
                       axis=1).reshape(1, _A * 4, _H, _W)
    biw_out = jnp.broadcast_to(biw.reshape(_A, 1, hw),
                               (_A, 4, hw)).reshape(1, _A * 4, _H, _W)
    bow_out = jnp.broadcast_to(bow.reshape(_A, 1, hw),
                               (_A, 4, hw)).reshape(1, _A * 4, _H, _W)
    return (labels_out, bt_out, biw_out, bow_out)


# speedup vs baseline: 30.7889x; 1.0506x over previous
import numpy as np
import jax
import jax.numpy as jnp
from jax.experimental import pallas as pl
from jax.experimental.pallas import tpu as pltpu

_FEAT_STRIDE = 16
_RPN_NEG = 0.3
_RPN_POS = 0.7
_RPN_BATCH = 256.0
_NUM_FG = 128.0
_H = 64
_W = 64
_A = 9
_N = _H * _W * _A
_R = _N // 128
_C = 128
_G = 100


def _np_all_anchors():
    base_size = 16.0
    xc = yc = 0.5 * (base_size - 1.0)
    size = base_size * base_size
    rows = []
    for r in (0.5, 1.0, 2.0):
        ws = np.round(np.sqrt(size / r))
        hs = np.round(ws * r)
        for s in (8.0, 16.0, 32.0):
            W_ = ws * s
            H_ = hs * s
            rows.append([xc - 0.5 * (W_ - 1.0), yc - 0.5 * (H_ - 1.0),
                         xc + 0.5 * (W_ - 1.0), yc + 0.5 * (H_ - 1.0)])
    base = np.array(rows, dtype=np.float32)
    sx = np.arange(_W, dtype=np.float32) * _FEAT_STRIDE
    sy = np.arange(_H, dtype=np.float32) * _FEAT_STRIDE
    SY, SX = np.meshgrid(sy, sx, indexing="ij")
    shifts = np.stack([SX.ravel(), SY.ravel(), SX.ravel(), SY.ravel()], axis=1)
    allv = (base[None, :, :] + shifts[:, None, :]).reshape(_N, 4)
    return allv.astype(np.float32)


_ANCHORS = _np_all_anchors()
_AM = np.ascontiguousarray(
    _ANCHORS.reshape(_H * _W, _A, 4).transpose(1, 0, 2).reshape(_N, 4))
_AX1 = np.ascontiguousarray(_AM[:, 0].reshape(_R, _C))
_AY1 = np.ascontiguousarray(_AM[:, 1].reshape(_R, _C))
_AX2 = np.ascontiguousarray(_AM[:, 2].reshape(_R, _C))
_AY2 = np.ascontiguousarray(_AM[:, 3].reshape(_R, _C))
_AX2P = _AX2 + 1.0
_AY2P = _AY2 + 1.0


def _body(gx1_ref, gy1_ref, gx2_ref, gy2_ref, areag_ref, im_ref,
          ax1_ref, ay1_ref, ax2_ref, ay2_ref, ax2p_ref, ay2p_ref,
          lab_ref, dx_ref, dy_ref, dw_ref, dh_ref, biw_ref, bow_ref,
          ovs_ref):
    ax1 = ax1_ref[...]
    ay1 = ay1_ref[...]
    ax2 = ax2_ref[...]
    ay2 = ay2_ref[...]
    ax2p = ax2p_ref[...]
    ay2p = ay2p_ref[...]
    im_h = im_ref[0]
    im_w = im_ref[1]
    inside = (ax1 >= 0.0) & (ay1 >= 0.0) & (ax2 < im_w) & (ay2 < im_h)
    area_a = (ax2p - ax1) * (ay2p - ay1)

    def pass1(j, carry):
        mx, g1, g2, g3, g4 = carry
        gx1 = gx1_ref[j]
        gy1 = gy1_ref[j]
        gx2 = gx2_ref[j]
        gy2 = gy2_ref[j]
        iw = jnp.minimum(ax2, gx2) - jnp.maximum(ax1, gx1) + 1.0
        ih = jnp.minimum(ay2, gy2) - jnp.maximum(ay1, gy1) + 1.0
        iw = jnp.maximum(iw, 0.0)
        ih = jnp.maximum(ih, 0.0)
        inter = iw * ih
        ov = jnp.where(inside, inter / (area_a + areag_ref[j] - inter), -1.0)
        ovs_ref[j] = ov
        upd = ov > mx
        mx = jnp.where(upd, ov, mx)
        g1 = jnp.where(upd, gx1, g1)
        g2 = jnp.where(upd, gy1, g2)
        g3 = jnp.where(upd, gx2, g3)
        g4 = jnp.where(upd, gy2, g4)
        return (mx, g1, g2, g3, g4)

    zero = jnp.zeros((_R, _C), jnp.float32)
    init = (jnp.full((_R, _C), -jnp.inf, jnp.float32), zero, zero, zero, zero)
    mx, g1, g2, g3, g4 = jax.lax.fori_loop(0, _G, pass1, init, unroll=8)

    ovs = ovs_ref[...]
    gtm = jnp.max(ovs, axis=(1, 2), keepdims=True)
    eq = jnp.where(ovs == gtm, 1.0, 0.0)
    isgt = (jnp.max(eq, axis=0) > 0.0) & inside

    lab = jnp.full((_R, _C), -1.0, jnp.float32)
    lab = jnp.where(mx < _RPN_NEG, 0.0, lab)
    lab = jnp.where(isgt, 1.0, lab)
    lab = jnp.where(mx >= _RPN_POS, 1.0, lab)
    lab = jnp.where(inside, lab, -1.0)

    _BR = (_H * _W) // _C
    ki = jax.lax.broadcasted_iota(jnp.int32, (_C, _C), 0)
    kj = jax.lax.broadcasted_iota(jnp.int32, (_C, _C), 1)
    upper = (ki <= kj).astype(jnp.float32)
    ri = jax.lax.broadcasted_iota(jnp.int32, (_R, _R), 0)
    rj = jax.lax.broadcasted_iota(jnp.int32, (_R, _R), 1)
    lower_blk = ((rj < ri) & (rj // _BR == ri // _BR)).astype(jnp.float32)

    def cumsum_orig_order(m):
        cw = jnp.dot(m, upper, preferred_element_type=jnp.float32)
        rs = cw[:, _C - 1:_C]
        pre = jnp.dot(lower_blk, rs, preferred_element_type=jnp.float32)
        s = cw + pre
        t = s[0:_BR, :]
        for a in range(1, _A):
            t = t + s[a * _BR:(a + 1) * _BR, :]
        out = [None] * _A
        u = jnp.zeros((_BR, _C), jnp.float32)
        for a in range(_A - 1, -1, -1):
            out[a] = t - u
            u = u + m[a * _BR:(a + 1) * _BR, :]
        return jnp.concatenate(out, axis=0)
    cumsum_flat = cumsum_orig_order

    fg = lab == 1.0
    cfg = cumsum_flat(fg.astype(jnp.float32))
    lab = jnp.where(fg & (cfg > _NUM_FG), -1.0, lab)
    num_bg = _RPN_BATCH - jnp.sum((lab == 1.0).astype(jnp.float32))
    bg = lab == 0.0
    cbg = cumsum_flat(bg.astype(jnp.float32))
    lab = jnp.where(bg & (cbg > num_bg), -1.0, lab)

    lab_ref[...] = lab

    ew = ax2p - ax1
    eh = ay2p - ay1
    ecx = ax1 + 0.5 * ew
    ecy = ay1 + 0.5 * eh
    gw = g3 - g1 + 1.0
    gh = g4 - g2 + 1.0
    gcx = g1 + 0.5 * gw
    gcy = g2 + 0.5 * gh
    dx_ref[...] = jnp.where(inside, (gcx - ecx) / ew, 0.0)
    dy_ref[...] = jnp.where(inside, (gcy - ecy) / eh, 0.0)
    dw_ref[...] = jnp.where(inside, jnp.log(gw / ew), 0.0)
    dh_ref[...] = jnp.where(inside, jnp.log(gh / eh), 0.0)

    biw_ref[...] = jnp.where(lab == 1.0, 1.0, 0.0)
    num_examples = jnp.sum((lab >= 0.0).astype(jnp.float32))
    bow_ref[...] = jnp.where(lab >= 0.0, 1.0 / num_examples, 0.0)


_plane = jax.ShapeDtypeStruct((_R, _C), jnp.float32)
_smem_spec = pl.BlockSpec(memory_space=pltpu.SMEM)

_call = pl.pallas_call(
    _body,
    out_shape=(_plane,) * 7,
    in_specs=[_smem_spec] * 6 + [pl.BlockSpec((_R, _C), lambda: (0, 0))] * 6,
    out_specs=tuple(pl.BlockSpec((_R, _C), lambda: (0, 0)) for _ in range(7)),
    scratch_shapes=[pltpu.VMEM((_G, _R, _C), jnp.float32)],
)


def kernel(rpn_cls_score, gt_boxes, im_info):
    gx1 = gt_boxes[:, 0]
    gy1 = gt_boxes[:, 1]
    gx2 = gt_boxes[:, 2]
    gy2 = gt_boxes[:, 3]
    areag = (gx2 - gx1 + 1.0) * (gy2 - gy1 + 1.0)
    im = im_info[0, :2]
    lab, dx, dy, dw, dh, biw, bow = _call(
        gx1, gy1, gx2, gy2, areag, im,
        _AX1, _AY1, _AX2, _AY2, _AX2P, _AY2P)

    labels_out = lab.reshape(1, 1, _A * _H, _W)

    hw = _H * _W
    bt_out = jnp.stack([dx.reshape(_A, hw), dy.reshape(_A, hw),
                        dw.reshape(_A, hw), dh.reshape(_A, hw)],
                       axis=1).reshape(1, _A * 4, _H, _W)
    biw_out = jnp.broadcast_to(biw.reshape(_A, 1, hw),
                               (_A, 4, hw)).reshape(1, _A * 4, _H, _W)
    bow_out = jnp.broadcast_to(bow.reshape(_A, 1, hw),
                               (_A, 4, hw)).reshape(1, _A * 4, _H, _W)
    return (labels_out, bt_out, biw_out, bow_out)


# speedup vs baseline: 31.3981x; 1.0198x over previous
import numpy as np
import jax
import jax.numpy as jnp
from jax.experimental import pallas as pl
from jax.experimental.pallas import tpu as pltpu

_FEAT_STRIDE = 16
_RPN_NEG = 0.3
_RPN_POS = 0.7
_RPN_BATCH = 256.0
_NUM_FG = 128.0
_H = 64
_W = 64
_A = 9
_N = _H * _W * _A
_R = _N // 128
_C = 128
_G = 100


def _np_all_anchors():
    base_size = 16.0
    xc = yc = 0.5 * (base_size - 1.0)
    size = base_size * base_size
    rows = []
    for r in (0.5, 1.0, 2.0):
        ws = np.round(np.sqrt(size / r))
        hs = np.round(ws * r)
        for s in (8.0, 16.0, 32.0):
            W_ = ws * s
            H_ = hs * s
            rows.append([xc - 0.5 * (W_ - 1.0), yc - 0.5 * (H_ - 1.0),
                         xc + 0.5 * (W_ - 1.0), yc + 0.5 * (H_ - 1.0)])
    base = np.array(rows, dtype=np.float32)
    sx = np.arange(_W, dtype=np.float32) * _FEAT_STRIDE
    sy = np.arange(_H, dtype=np.float32) * _FEAT_STRIDE
    SY, SX = np.meshgrid(sy, sx, indexing="ij")
    shifts = np.stack([SX.ravel(), SY.ravel(), SX.ravel(), SY.ravel()], axis=1)
    allv = (base[None, :, :] + shifts[:, None, :]).reshape(_N, 4)
    return allv.astype(np.float32)


_ANCHORS = _np_all_anchors()
_AM = np.ascontiguousarray(
    _ANCHORS.reshape(_H * _W, _A, 4).transpose(1, 0, 2).reshape(_N, 4))
_AX1 = np.ascontiguousarray(_AM[:, 0].reshape(_R, _C))
_AY1 = np.ascontiguousarray(_AM[:, 1].reshape(_R, _C))
_AX2 = np.ascontiguousarray(_AM[:, 2].reshape(_R, _C))
_AY2 = np.ascontiguousarray(_AM[:, 3].reshape(_R, _C))
_AX2P = _AX2 + 1.0
_AY2P = _AY2 + 1.0


def _body(gx1_ref, gy1_ref, gx2_ref, gy2_ref, areag_ref, im_ref,
          ax1_ref, ay1_ref, ax2_ref, ay2_ref, ax2p_ref, ay2p_ref,
          lab_ref, dx_ref, dy_ref, dw_ref, dh_ref, biw_ref, bow_ref,
          ovs_ref):
    ax1 = ax1_ref[...]
    ay1 = ay1_ref[...]
    ax2 = ax2_ref[...]
    ay2 = ay2_ref[...]
    ax2p = ax2p_ref[...]
    ay2p = ay2p_ref[...]
    im_h = im_ref[0]
    im_w = im_ref[1]
    inside = (ax1 >= 0.0) & (ay1 >= 0.0) & (ax2 < im_w) & (ay2 < im_h)
    area_a = (ax2p - ax1) * (ay2p - ay1)

    def pass1(j, carry):
        mx, g1, g2, g3, g4 = carry
        gx1 = gx1_ref[j]
        gy1 = gy1_ref[j]
        gx2 = gx2_ref[j]
        gy2 = gy2_ref[j]
        iw = jnp.minimum(ax2, gx2) - jnp.maximum(ax1, gx1) + 1.0
        ih = jnp.minimum(ay2, gy2) - jnp.maximum(ay1, gy1) + 1.0
        iw = jnp.maximum(iw, 0.0)
        ih = jnp.maximum(ih, 0.0)
        inter = iw * ih
        ov = jnp.where(inside, inter / (area_a + areag_ref[j] - inter), -1.0)
        ovs_ref[j] = ov
        upd = ov > mx
        mx = jnp.where(upd, ov, mx)
        g1 = jnp.where(upd, gx1, g1)
        g2 = jnp.where(upd, gy1, g2)
        g3 = jnp.where(upd, gx2, g3)
        g4 = jnp.where(upd, gy2, g4)
        return (mx, g1, g2, g3, g4)

    zero = jnp.zeros((_R, _C), jnp.float32)
    init = (jnp.full((_R, _C), -jnp.inf, jnp.float32), zero, zero, zero, zero)
    mx, g1, g2, g3, g4 = jax.lax.fori_loop(0, _G, pass1, init, unroll=16)

    ovs = ovs_ref[...]
    gtm = jnp.max(ovs, axis=(1, 2), keepdims=True)
    eq = jnp.where(ovs == gtm, 1.0, 0.0)
    isgt = (jnp.max(eq, axis=0) > 0.0) & inside

    lab = jnp.full((_R, _C), -1.0, jnp.float32)
    lab = jnp.where(mx < _RPN_NEG, 0.0, lab)
    lab = jnp.where(isgt, 1.0, lab)
    lab = jnp.where(mx >= _RPN_POS, 1.0, lab)
    lab = jnp.where(inside, lab, -1.0)

    _BR = (_H * _W) // _C
    ki = jax.lax.broadcasted_iota(jnp.int32, (_C, _C), 0)
    kj = jax.lax.broadcasted_iota(jnp.int32, (_C, _C), 1)
    upper = (ki <= kj).astype(jnp.float32)
    ri = jax.lax.broadcasted_iota(jnp.int32, (_R, _R), 0)
    rj = jax.lax.broadcasted_iota(jnp.int32, (_R, _R), 1)
    lower_blk = ((rj < ri) & (rj // _BR == ri // _BR)).astype(jnp.float32)

    def cumsum_orig_order(m):
        cw = jnp.dot(m, upper, preferred_element_type=jnp.float32)
        rs = cw[:, _C - 1:_C]
        pre = jnp.dot(lower_blk, rs, preferred_element_type=jnp.float32)
        s = cw + pre
        t = s[0:_BR, :]
        for a in range(1, _A):
            t = t + s[a * _BR:(a + 1) * _BR, :]
        out = [None] * _A
        u = jnp.zeros((_BR, _C), jnp.float32)
        for a in range(_A - 1, -1, -1):
            out[a] = t - u
            u = u + m[a * _BR:(a + 1) * _BR, :]
        return jnp.concatenate(out, axis=0)
    cumsum_flat = cumsum_orig_order

    fg = lab == 1.0
    cfg = cumsum_flat(fg.astype(jnp.float32))
    lab = jnp.where(fg & (cfg > _NUM_FG), -1.0, lab)
    num_bg = _RPN_BATCH - jnp.sum((lab == 1.0).astype(jnp.float32))
    bg = lab == 0.0
    cbg = cumsum_flat(bg.astype(jnp.float32))
    lab = jnp.where(bg & (cbg > num_bg), -1.0, lab)

    lab_ref[...] = lab

    ew = ax2p - ax1
    eh = ay2p - ay1
    ecx = ax1 + 0.5 * ew
    ecy = ay1 + 0.5 * eh
    gw = g3 - g1 + 1.0
    gh = g4 - g2 + 1.0
    gcx = g1 + 0.5 * gw
    gcy = g2 + 0.5 * gh
    dx_ref[...] = jnp.where(inside, (gcx - ecx) / ew, 0.0)
    dy_ref[...] = jnp.where(inside, (gcy - ecy) / eh, 0.0)
    dw_ref[...] = jnp.where(inside, jnp.log(gw / ew), 0.0)
    dh_ref[...] = jnp.where(inside, jnp.log(gh / eh), 0.0)

    biw_ref[...] = jnp.where(lab == 1.0, 1.0, 0.0)
    num_examples = jnp.sum((lab >= 0.0).astype(jnp.float32))
    bow_ref[...] = jnp.where(lab >= 0.0, 1.0 / num_examples, 0.0)


_plane = jax.ShapeDtypeStruct((_R, _C), jnp.float32)
_smem_spec = pl.BlockSpec(memory_space=pltpu.SMEM)

_call = pl.pallas_call(
    _body,
    out_shape=(_plane,) * 7,
    in_specs=[_smem_spec] * 6 + [pl.BlockSpec((_R, _C), lambda: (0, 0))] * 6,
    out_specs=tuple(pl.BlockSpec((_R, _C), lambda: (0, 0)) for _ in range(7)),
    scratch_shapes=[pltpu.VMEM((_G, _R, _C), jnp.float32)],
)


def kernel(rpn_cls_score, gt_boxes, im_info):
    gx1 = gt_boxes[:, 0]
    gy1 = gt_boxes[:, 1]
    gx2 = gt_boxes[:, 2]
    gy2 = gt_boxes[:, 3]
    areag = (gx2 - gx1 + 1.0) * (gy2 - gy1 + 1.0)
    im = im_info[0, :2]
    lab, dx, dy, dw, dh, biw, bow = _call(
        gx1, gy1, gx2, gy2, areag, im,
        _AX1, _AY1, _AX2, _AY2, _AX2P, _AY2P)

    labels_out = lab.reshape(1, 1, _A * _H, _W)

    hw = _H * _W
    bt_out = jnp.stack([dx.reshape(_A, hw), dy.reshape(_A, hw),
                        dw.reshape(_A, hw), dh.reshape(_A, hw)],
                       axis=1).reshape(1, _A * 4, _H, _W)
    biw_out = jnp.broadcast_to(biw.reshape(_A, 1, hw),
                               (_A, 4, hw)).reshape(1, _A * 4, _H, _W)
    bow_out = jnp.broadcast_to(bow.reshape(_A, 1, hw),
                               (_A, 4, hw)).reshape(1, _A * 4, _H, _W)
    return (labels_out, bt_out, biw_out, bow_out)


# speedup vs baseline: 31.5446x; 1.0047x over previous
import numpy as np
import jax
import jax.numpy as jnp
from jax.experimental import pallas as pl
from jax.experimental.pallas import tpu as pltpu

_FEAT_STRIDE = 16
_RPN_NEG = 0.3
_RPN_POS = 0.7
_RPN_BATCH = 256.0
_NUM_FG = 128.0
_H = 64
_W = 64
_A = 9
_N = _H * _W * _A
_R = _N // 128
_C = 128
_G = 100


def _np_all_anchors():
    base_size = 16.0
    xc = yc = 0.5 * (base_size - 1.0)
    size = base_size * base_size
    rows = []
    for r in (0.5, 1.0, 2.0):
        ws = np.round(np.sqrt(size / r))
        hs = np.round(ws * r)
        for s in (8.0, 16.0, 32.0):
            W_ = ws * s
            H_ = hs * s
            rows.append([xc - 0.5 * (W_ - 1.0), yc - 0.5 * (H_ - 1.0),
                         xc + 0.5 * (W_ - 1.0), yc + 0.5 * (H_ - 1.0)])
    base = np.array(rows, dtype=np.float32)
    sx = np.arange(_W, dtype=np.float32) * _FEAT_STRIDE
    sy = np.arange(_H, dtype=np.float32) * _FEAT_STRIDE
    SY, SX = np.meshgrid(sy, sx, indexing="ij")
    shifts = np.stack([SX.ravel(), SY.ravel(), SX.ravel(), SY.ravel()], axis=1)
    allv = (base[None, :, :] + shifts[:, None, :]).reshape(_N, 4)
    return allv.astype(np.float32)


_ANCHORS = _np_all_anchors()
_AM = np.ascontiguousarray(
    _ANCHORS.reshape(_H * _W, _A, 4).transpose(1, 0, 2).reshape(_N, 4))
_AX1 = np.ascontiguousarray(_AM[:, 0].reshape(_R, _C))
_AY1 = np.ascontiguousarray(_AM[:, 1].reshape(_R, _C))
_AX2 = np.ascontiguousarray(_AM[:, 2].reshape(_R, _C))
_AY2 = np.ascontiguousarray(_AM[:, 3].reshape(_R, _C))
_AX2P = _AX2 + 1.0
_AY2P = _AY2 + 1.0


def _body(gx1_ref, gy1_ref, gx2_ref, gy2_ref, areag_ref, im_ref,
          ax1_ref, ay1_ref, ax2_ref, ay2_ref, ax2p_ref, ay2p_ref,
          lab_ref, dx_ref, dy_ref, dw_ref, dh_ref, biw_ref, bow_ref,
          ovs_ref):
    ax1 = ax1_ref[...]
    ay1 = ay1_ref[...]
    ax2 = ax2_ref[...]
    ay2 = ay2_ref[...]
    ax2p = ax2p_ref[...]
    ay2p = ay2p_ref[...]
    im_h = im_ref[0]
    im_w = im_ref[1]
    inside = (ax1 >= 0.0) & (ay1 >= 0.0) & (ax2 < im_w) & (ay2 < im_h)
    area_a = (ax2p - ax1) * (ay2p - ay1)

    def pass1(j, carry):
        mx, g1, g2, g3, g4 = carry
        gx1 = gx1_ref[j]
        gy1 = gy1_ref[j]
        gx2 = gx2_ref[j]
        gy2 = gy2_ref[j]
        iw = jnp.minimum(ax2, gx2) - jnp.maximum(ax1, gx1) + 1.0
        ih = jnp.minimum(ay2, gy2) - jnp.maximum(ay1, gy1) + 1.0
        iw = jnp.maximum(iw, 0.0)
        ih = jnp.maximum(ih, 0.0)
        inter = iw * ih
        ov = jnp.where(inside, inter / (area_a + areag_ref[j] - inter), -1.0)
        ovs_ref[j] = ov
        upd = ov > mx
        mx = jnp.where(upd, ov, mx)
        g1 = jnp.where(upd, gx1, g1)
        g2 = jnp.where(upd, gy1, g2)
        g3 = jnp.where(upd, gx2, g3)
        g4 = jnp.where(upd, gy2, g4)
        return (mx, g1, g2, g3, g4)

    zero = jnp.zeros((_R, _C), jnp.float32)
    init = (jnp.full((_R, _C), -jnp.inf, jnp.float32), zero, zero, zero, zero)
    mx, g1, g2, g3, g4 = jax.lax.fori_loop(0, _G, pass1, init, unroll=25)

    ovs = ovs_ref[...]
    gtm = jnp.max(ovs, axis=(1, 2), keepdims=True)
    eq = jnp.where(ovs == gtm, 1.0, 0.0)
    isgt = (jnp.max(eq, axis=0) > 0.0) & inside

    lab = jnp.full((_R, _C), -1.0, jnp.float32)
    lab = jnp.where(mx < _RPN_NEG, 0.0, lab)
    lab = jnp.where(isgt, 1.0, lab)
    lab = jnp.where(mx >= _RPN_POS, 1.0, lab)
    lab = jnp.where(inside, lab, -1.0)

    _BR = (_H * _W) // _C
    ki = jax.lax.broadcasted_iota(jnp.int32, (_C, _C), 0)
    kj = jax.lax.broadcasted_iota(jnp.int32, (_C, _C), 1)
    upper = (ki <= kj).astype(jnp.float32)
    ri = jax.lax.broadcasted_iota(jnp.int32, (_R, _R), 0)
    rj = jax.lax.broadcasted_iota(jnp.int32, (_R, _R), 1)
    lower_blk = ((rj < ri) & (rj // _BR == ri // _BR)).astype(jnp.float32)

    def cumsum_orig_order(m):
        cw = jnp.dot(m, upper, preferred_element_type=jnp.float32)
        rs = cw[:, _C - 1:_C]
        pre = jnp.dot(lower_blk, rs, preferred_element_type=jnp.float32)
        s = cw + pre
        t = s[0:_BR, :]
        for a in range(1, _A):
            t = t + s[a * _BR:(a + 1) * _BR, :]
        out = [None] * _A
        u = jnp.zeros((_BR, _C), jnp.float32)
        for a in range(_A - 1, -1, -1):
            out[a] = t - u
            u = u + m[a * _BR:(a + 1) * _BR, :]
        return jnp.concatenate(out, axis=0)
    cumsum_flat = cumsum_orig_order

    fg = lab == 1.0
    cfg = cumsum_flat(fg.astype(jnp.float32))
    lab = jnp.where(fg & (cfg > _NUM_FG), -1.0, lab)
    num_bg = _RPN_BATCH - jnp.sum((lab == 1.0).astype(jnp.float32))
    bg = lab == 0.0
    cbg = cumsum_flat(bg.astype(jnp.float32))
    lab = jnp.where(bg & (cbg > num_bg), -1.0, lab)

    lab_ref[...] = lab

    ew = ax2p - ax1
    eh = ay2p - ay1
    ecx = ax1 + 0.5 * ew
    ecy = ay1 + 0.5 * eh
    gw = g3 - g1 + 1.0
    gh = g4 - g2 + 1.0
    gcx = g1 + 0.5 * gw
    gcy = g2 + 0.5 * gh
    dx_ref[...] = jnp.where(inside, (gcx - ecx) / ew, 0.0)
    dy_ref[...] = jnp.where(inside, (gcy - ecy) / eh, 0.0)
    dw_ref[...] = jnp.where(inside, jnp.log(gw / ew), 0.0)
    dh_ref[...] = jnp.where(inside, jnp.log(gh / eh), 0.0)

    biw_ref[...] = jnp.where(lab == 1.0, 1.0, 0.0)
    num_examples = jnp.sum((lab >= 0.0).astype(jnp.float32))
    bow_ref[...] = jnp.where(lab >= 0.0, 1.0 / num_examples, 0.0)


_plane = jax.ShapeDtypeStruct((_R, _C), jnp.float32)
_smem_spec = pl.BlockSpec(memory_space=pltpu.SMEM)

_call = pl.pallas_call(
    _body,
    out_shape=(_plane,) * 7,
    in_specs=[_smem_spec] * 6 + [pl.BlockSpec((_R, _C), lambda: (0, 0))] * 6,
    out_specs=tuple(pl.BlockSpec((_R, _C), lambda: (0, 0)) for _ in range(7)),
    scratch_shapes=[pltpu.VMEM((_G, _R, _C), jnp.float32)],
)


def kernel(rpn_cls_score, gt_boxes, im_info):
    gx1 = gt_boxes[:, 0]
    gy1 = gt_boxes[:, 1]
    gx2 = gt_boxes[:, 2]
    gy2 = gt_boxes[:, 3]
    areag = (gx2 - gx1 + 1.0) * (gy2 - gy1 + 1.0)
    im = im_info[0, :2]
    lab, dx, dy, dw, dh, biw, bow = _call(
        gx1, gy1, gx2, gy2, areag, im,
        _AX1, _AY1, _AX2, _AY2, _AX2P, _AY2P)

    labels_out = lab.reshape(1, 1, _A * _H, _W)

    hw = _H * _W
    bt_out = jnp.stack([dx.reshape(_A, hw), dy.reshape(_A, hw),
                        dw.reshape(_A, hw), dh.reshape(_A, hw)],
                       axis=1).reshape(1, _A * 4, _H, _W)
    biw_out = jnp.broadcast_to(biw.reshape(_A, 1, hw),
                               (_A, 4, hw)).reshape(1, _A * 4, _H, _W)
    bow_out = jnp.broadcast_to(bow.reshape(_A, 1, hw),
                               (_A, 4, hw)).reshape(1, _A * 4, _H, _W)
    return (labels_out, bt_out, biw_out, bow_out)
